# baseline (speedup 1.0000x reference)
import functools

import jax
import jax.numpy as jnp
from jax import lax
from jax.experimental import pallas as pl
from jax.experimental.pallas import tpu as pltpu
from jax.experimental.pallas import tpu_sc as plsc

_H = 64
_NC = 2
_NS = 16
_NW = _NC * _NS
_CHUNK = 400
_BLK = 4096
_SC_ROWS = 409600


def _proj_body(emb_ref, w_ref, b_ref, out_ref):
    out_ref[...] = lax.dot_general(
        emb_ref[...], w_ref[...], (((1,), (1,)), ((), ())),
        preferred_element_type=jnp.float32) + b_ref[...]


def _tc_body(p_ref, x_ref, t_ref, out_ref):
    tv = t_ref[0, 0, :]
    onehot = (tv[:, None] == lax.broadcasted_iota(jnp.int32, (1, _H), 1)
              ).astype(jnp.float32)
    e = lax.dot_general(onehot, p_ref[...], (((1,), (0,)), ((), ())),
                        preferred_element_type=jnp.float32)
    out_ref[...] = x_ref[...] + e


def _tc_call(p, x2, t1):
    n_rows = x2.shape[0]
    n_blocks = n_rows // _BLK
    return pl.pallas_call(
        _tc_body,
        grid=(n_blocks,),
        in_specs=[
            pl.BlockSpec((_H, _H), lambda i: (0, 0)),
            pl.BlockSpec((_BLK, _H), lambda i: (i, 0)),
            pl.BlockSpec((1, 1, _BLK), lambda i: (i, 0, 0)),
        ],
        out_specs=pl.BlockSpec((_BLK, _H), lambda i: (i, 0)),
        out_shape=jax.ShapeDtypeStruct((n_rows, _H), jnp.float32),
    )(p, x2, t1.reshape(n_blocks, 1, _BLK))


def _make_sc_call(n_rows: int):
    rows_per_w = n_rows // _NW
    n_chunks = rows_per_w // _CHUNK
    assert n_chunks % 2 == 0

    def _sc_body(p_hbm, x_hbm, t_hbm, out_hbm,
                 p_v, t0, t1, buf0, buf1, isem0, isem1, osem0, osem1):
        wid = lax.axis_index("s") * _NC + lax.axis_index("c")
        base = wid * rows_per_w
        pltpu.sync_copy(p_hbm, p_v)
        slots = ((t0, buf0, isem0, osem0), (t1, buf1, isem1, osem1))

        def start_in(g, sl):
            start = base + g * _CHUNK
            t_v, buf, isem, _ = slots[sl]
            pltpu.async_copy(t_hbm.at[pl.ds(start, _CHUNK)], t_v, isem)
            pltpu.async_copy(x_hbm.at[pl.ds(start, _CHUNK)], buf, isem)

        def wait_in(sl):
            t_v, buf, isem, _ = slots[sl]
            pltpu.make_async_copy(t_hbm.at[pl.ds(0, _CHUNK)], t_v, isem).wait()
            pltpu.make_async_copy(x_hbm.at[pl.ds(0, _CHUNK)], buf, isem).wait()

        def start_out(g, sl):
            start = base + g * _CHUNK
            _, buf, _, osem = slots[sl]
            pltpu.async_copy(buf, out_hbm.at[pl.ds(start, _CHUNK)], osem)

        def wait_out(sl):
            _, buf, _, osem = slots[sl]
            pltpu.make_async_copy(
                buf, out_hbm.at[pl.ds(0, _CHUNK)], osem).wait()

        def compute(sl):
            t_v, buf, _, _ = slots[sl]

            @plsc.parallel_loop(0, _CHUNK, 16)
            def _rows(i):
                tvec = t_v[pl.ds(i, 16)]
                for h in range(2):
                    vals = []
                    for k in range(8 * h, 8 * h + 8):
                        ti = tvec[k]
                        vals.append([p_v[ti, pl.ds(cg * 16, 16)]
                                     for cg in range(_H // 16)])
                    for k in range(8 * h, 8 * h + 8):
                        for cg in range(_H // 16):
                            plsc.addupdate(
                                buf.at[i + k, pl.ds(cg * 16, 16)],
                                vals[k - 8 * h][cg])

        start_in(0, 0)
        start_in(1, 1)

        def pair_body(gp, carry):
            g = gp * 2
            for sl in range(2):
                wait_in(sl)
                compute(sl)
                start_out(g + sl, sl)

                @pl.when(gp + 1 < n_chunks // 2)
                def _():
                    wait_out(sl)
                    start_in(g + sl + 2, sl)

            return carry

        lax.fori_loop(0, n_chunks // 2, pair_body, 0)
        wait_out(0)
        wait_out(1)

    return pl.kernel(
        _sc_body,
        out_type=jax.ShapeDtypeStruct((n_rows, _H), jnp.float32),
        mesh=plsc.VectorSubcoreMesh(core_axis_name="c", subcore_axis_name="s"),
        scratch_types=[
            pltpu.VMEM((_H, _H), jnp.float32),
            pltpu.VMEM((_CHUNK,), jnp.int32),
            pltpu.VMEM((_CHUNK,), jnp.int32),
            pltpu.VMEM((_CHUNK, _H), jnp.float32),
            pltpu.VMEM((_CHUNK, _H), jnp.float32),
            pltpu.SemaphoreType.DMA,
            pltpu.SemaphoreType.DMA,
            pltpu.SemaphoreType.DMA,
            pltpu.SemaphoreType.DMA,
        ],
    )


def kernel(x, t, emb_table, W, b):
    batch, hist, h = x.shape
    n_rows = batch * hist
    p = pl.pallas_call(
        _proj_body,
        out_shape=jax.ShapeDtypeStruct((_H, _H), jnp.float32),
    )(emb_table, W, b.reshape(1, _H))
    x2 = x.reshape(n_rows, h)
    t1 = t.reshape(n_rows)
    out_sc = _make_sc_call(_SC_ROWS)(p, x2[:_SC_ROWS], t1[:_SC_ROWS])
    out_tc = _tc_call(p, x2[_SC_ROWS:], t1[_SC_ROWS:])
    out = jnp.concatenate([out_sc, out_tc], axis=0)
    return out.reshape(x.shape)


# speedup vs baseline: 1.3102x; 1.3102x over previous
import functools

import jax
import jax.numpy as jnp
from jax import lax
from jax.experimental import pallas as pl
from jax.experimental.pallas import tpu as pltpu
from jax.experimental.pallas import tpu_sc as plsc

_H = 64
_NC = 2
_NS = 16
_NW = _NC * _NS
_CHUNK = 400
_BLK = 4096
_SC_ROWS = 307200


def _proj_body(emb_ref, w_ref, b_ref, out_ref):
    out_ref[...] = lax.dot_general(
        emb_ref[...], w_ref[...], (((1,), (1,)), ((), ())),
        preferred_element_type=jnp.float32) + b_ref[...]


def _tc_body(p_ref, x_ref, t_ref, out_ref):
    tv = t_ref[0, 0, :]
    onehot = (tv[:, None] == lax.broadcasted_iota(jnp.int32, (1, _H), 1)
              ).astype(jnp.float32)
    e = lax.dot_general(onehot, p_ref[...], (((1,), (0,)), ((), ())),
                        preferred_element_type=jnp.float32)
    out_ref[...] = x_ref[...] + e


def _tc_call(p, x2, t1):
    n_rows = x2.shape[0]
    blk0 = _SC_ROWS // _BLK
    n_blocks = n_rows // _BLK - blk0
    return pl.pallas_call(
        _tc_body,
        grid=(n_blocks,),
        in_specs=[
            pl.BlockSpec((_H, _H), lambda i: (0, 0)),
            pl.BlockSpec((_BLK, _H), lambda i: (blk0 + i, 0)),
            pl.BlockSpec((1, 1, _BLK), lambda i: (blk0 + i, 0, 0)),
        ],
        out_specs=pl.BlockSpec((_BLK, _H), lambda i: (i, 0)),
        out_shape=jax.ShapeDtypeStruct((n_rows - _SC_ROWS, _H), jnp.float32),
    )(p, x2, t1.reshape(n_rows // _BLK, 1, _BLK))


def _make_sc_call(n_rows: int, sc_rows: int):
    rows_per_w = sc_rows // _NW
    n_chunks = rows_per_w // _CHUNK
    assert n_chunks % 2 == 0

    def _sc_body(p_hbm, x_hbm, t_hbm, out_hbm,
                 p_v, t0, t1, buf0, buf1, isem0, isem1, osem0, osem1):
        wid = lax.axis_index("s") * _NC + lax.axis_index("c")
        base = wid * rows_per_w
        pltpu.sync_copy(p_hbm, p_v)
        slots = ((t0, buf0, isem0, osem0), (t1, buf1, isem1, osem1))

        def start_in(g, sl):
            start = base + g * _CHUNK
            t_v, buf, isem, _ = slots[sl]
            pltpu.async_copy(t_hbm.at[pl.ds(start, _CHUNK)], t_v, isem)
            pltpu.async_copy(x_hbm.at[pl.ds(start, _CHUNK)], buf, isem)

        def wait_in(sl):
            t_v, buf, isem, _ = slots[sl]
            pltpu.make_async_copy(t_hbm.at[pl.ds(0, _CHUNK)], t_v, isem).wait()
            pltpu.make_async_copy(x_hbm.at[pl.ds(0, _CHUNK)], buf, isem).wait()

        def start_out(g, sl):
            start = base + g * _CHUNK
            _, buf, _, osem = slots[sl]
            pltpu.async_copy(buf, out_hbm.at[pl.ds(start, _CHUNK)], osem)

        def wait_out(sl):
            _, buf, _, osem = slots[sl]
            pltpu.make_async_copy(
                buf, out_hbm.at[pl.ds(0, _CHUNK)], osem).wait()

        def compute(sl):
            t_v, buf, _, _ = slots[sl]

            @plsc.parallel_loop(0, _CHUNK, 16)
            def _rows(i):
                tvec = t_v[pl.ds(i, 16)]
                for h in range(2):
                    vals = []
                    for k in range(8 * h, 8 * h + 8):
                        ti = tvec[k]
                        vals.append([p_v[ti, pl.ds(cg * 16, 16)]
                                     for cg in range(_H // 16)])
                    for k in range(8 * h, 8 * h + 8):
                        for cg in range(_H // 16):
                            plsc.addupdate(
                                buf.at[i + k, pl.ds(cg * 16, 16)],
                                vals[k - 8 * h][cg])

        start_in(0, 0)
        start_in(1, 1)

        def pair_body(gp, carry):
            g = gp * 2
            for sl in range(2):
                wait_in(sl)
                compute(sl)
                start_out(g + sl, sl)

                @pl.when(gp + 1 < n_chunks // 2)
                def _():
                    wait_out(sl)
                    start_in(g + sl + 2, sl)

            return carry

        lax.fori_loop(0, n_chunks // 2, pair_body, 0)
        wait_out(0)
        wait_out(1)

    return pl.kernel(
        _sc_body,
        out_type=jax.ShapeDtypeStruct((n_rows, _H), jnp.float32),
        mesh=plsc.VectorSubcoreMesh(core_axis_name="c", subcore_axis_name="s"),
        scratch_types=[
            pltpu.VMEM((_H, _H), jnp.float32),
            pltpu.VMEM((_CHUNK,), jnp.int32),
            pltpu.VMEM((_CHUNK,), jnp.int32),
            pltpu.VMEM((_CHUNK, _H), jnp.float32),
            pltpu.VMEM((_CHUNK, _H), jnp.float32),
            pltpu.SemaphoreType.DMA,
            pltpu.SemaphoreType.DMA,
            pltpu.SemaphoreType.DMA,
            pltpu.SemaphoreType.DMA,
        ],
    )


def kernel(x, t, emb_table, W, b):
    batch, hist, h = x.shape
    n_rows = batch * hist
    p = pl.pallas_call(
        _proj_body,
        out_shape=jax.ShapeDtypeStruct((_H, _H), jnp.float32),
    )(emb_table, W, b.reshape(1, _H))
    x2 = x.reshape(n_rows, h)
    t1 = t.reshape(n_rows)
    out_sc = _make_sc_call(n_rows, _SC_ROWS)(p, x2, t1)
    out_tc = _tc_call(p, x2, t1)
    out = lax.dynamic_update_slice(out_sc, out_tc, (_SC_ROWS, 0))
    return out.reshape(x.shape)
